# baseline (speedup 1.0000x reference)
import functools

import jax
import jax.numpy as jnp
from jax import lax
from jax.experimental import pallas as pl
from jax.experimental.pallas import tpu as pltpu
from jax.experimental.pallas import tpu_sc as plsc

NUM_SAMPLES = 1000000
BATCH = 16384

_INFO = plsc.get_sparse_core_info()
_NC, _NS, _L = _INFO.num_cores, _INFO.num_subcores, _INFO.num_lanes
_NW = _NC * _NS
_BPW = BATCH // _NW
_CHUNK = 128
_NCHUNK = _BPW // _CHUNK
_UNROLL = 4

_mesh = plsc.VectorSubcoreMesh(core_axis_name="c", subcore_axis_name="s")


@functools.partial(
    pl.kernel,
    mesh=_mesh,
    out_type=jax.ShapeDtypeStruct((BATCH,), jnp.float32),
    scratch_types=[
        pltpu.VMEM((_BPW,), jnp.int32),
        pltpu.VMEM((_BPW,), jnp.float32),
    ] + [pltpu.SemaphoreType.DMA] * _NCHUNK,
)
def _gather_tanh(table_hbm, idx_hbm, out_hbm, idx_v, vals_v, *sems):
    wid = lax.axis_index("s") * _NC + lax.axis_index("c")
    base = wid * _BPW
    pltpu.sync_copy(idx_hbm.at[pl.ds(base, _BPW)], idx_v)
    gathers = [
        pltpu.async_copy(table_hbm.at[idx_v.at[pl.ds(c * _CHUNK, _CHUNK)]],
                         vals_v.at[pl.ds(c * _CHUNK, _CHUNK)], sems[c])
        for c in range(_NCHUNK)
    ]
    out_cp = []
    for c in range(_NCHUNK):
        gathers[c].wait()

        def _tanh_step(i, _, c=c):
            for u in range(_UNROLL):
                sl = pl.ds(c * _CHUNK + (i * _UNROLL + u) * _L, _L)
                x = vals_v[sl]
                e = jnp.exp(jnp.abs(x) * -2.0)
                vals_v[sl] = jnp.sign(x) * ((1.0 - e) / (1.0 + e))
            return _

        lax.fori_loop(0, _CHUNK // (_L * _UNROLL), _tanh_step, 0)
        out_cp.append(
            pltpu.async_copy(vals_v.at[pl.ds(c * _CHUNK, _CHUNK)],
                             out_hbm.at[pl.ds(base + c * _CHUNK, _CHUNK)],
                             sems[c]))
    for c in out_cp:
        c.wait()


def kernel(sample_indices, sample_logits):
    return _gather_tanh(sample_logits, sample_indices)


# speedup vs baseline: 1.0233x; 1.0233x over previous
import functools

import jax
import jax.numpy as jnp
from jax import lax
from jax.experimental import pallas as pl
from jax.experimental.pallas import tpu as pltpu
from jax.experimental.pallas import tpu_sc as plsc

NUM_SAMPLES = 1000000
BATCH = 16384

_INFO = plsc.get_sparse_core_info()
_NC, _NS, _L = _INFO.num_cores, _INFO.num_subcores, _INFO.num_lanes
_NW = _NC * _NS
_BPW = BATCH // _NW
_CHUNK = 128
_NCHUNK = _BPW // _CHUNK
_UNROLL = 4

_mesh = plsc.VectorSubcoreMesh(core_axis_name="c", subcore_axis_name="s")


@functools.partial(
    pl.kernel,
    mesh=_mesh,
    out_type=jax.ShapeDtypeStruct((BATCH,), jnp.float32),
    scratch_types=[
        pltpu.VMEM((_BPW,), jnp.int32),
        pltpu.VMEM((_BPW,), jnp.float32),
    ] + [pltpu.SemaphoreType.DMA],
)
def _gather_tanh(table_hbm, idx_hbm, out_hbm, idx_v, vals_v, *sems):
    wid = lax.axis_index("s") * _NC + lax.axis_index("c")
    base = wid * _BPW
    pltpu.sync_copy(idx_hbm.at[pl.ds(base, _BPW)], idx_v)
    pltpu.async_copy(table_hbm.at[idx_v], vals_v, sems[0]).wait()

    def _tanh_step(i, _):
        for u in range(_UNROLL):
            sl = pl.ds((i * _UNROLL + u) * _L, _L)
            x = vals_v[sl]
            vals_v[sl] = 1.0 - 2.0 / (jnp.exp(x * 2.0) + 1.0)
        return _

    lax.fori_loop(0, _BPW // (_L * _UNROLL), _tanh_step, 0)
    pltpu.sync_copy(vals_v, out_hbm.at[pl.ds(base, _BPW)])


def kernel(sample_indices, sample_logits):
    return _gather_tanh(sample_logits, sample_indices)
